# baseline (speedup 1.0000x reference)
import jax
import jax.numpy as jnp
from jax import lax
from jax.experimental import pallas as pl
from jax.experimental.pallas import tpu as pltpu

_RPG = 64


def _fused_kernel(x_ref, w1_ref, b1_ref, w2_ref, b2_ref, out_ref, acc_ref,
                  *, rows_total):
    k = pl.program_id(0)
    steps = pl.num_programs(0)
    lanes = x_ref.shape[2]
    always_valid = rows_total - (steps - 1) * _RPG

    def sub_dot(t):
        return lax.dot_general(
            x_ref[:, t, :], w1_ref[:, t * lanes:(t + 1) * lanes],
            dimension_numbers=(((1,), (1,)), ((), ())),
            preferred_element_type=jnp.float32)

    acc = sub_dot(0)
    for t in range(1, _RPG):
        p = sub_dot(t)
        if t >= always_valid:
            p = jnp.where(k * _RPG + t < rows_total, p, 0.0)
        acc += p

    @pl.when(k == 0)
    def _init():
        acc_ref[...] = acc

    @pl.when(k > 0)
    def _accum():
        acc_ref[...] += acc

    @pl.when(k == steps - 1)
    def _finalize():
        hidden = acc_ref[...] + b1_ref[...]
        hidden = 1.0 / (1.0 + jnp.exp(-hidden))
        pred = lax.dot_general(
            hidden, w2_ref[...],
            dimension_numbers=(((1,), (1,)), ((), ())),
            preferred_element_type=jnp.float32) + b2_ref[...]
        out_ref[...] = pred.astype(out_ref.dtype)


def kernel(emb, w1, b1, w2, b2):
    import functools
    B, R, L = emb.shape
    H, K = w1.shape
    C = w2.shape[0]
    assert R * L == K, (emb.shape, w1.shape)
    steps = -(-R // _RPG)
    bk = _RPG * L

    return pl.pallas_call(
        functools.partial(_fused_kernel, rows_total=R),
        out_shape=jax.ShapeDtypeStruct((B, C), emb.dtype),
        grid=(steps,),
        in_specs=[
            pl.BlockSpec((B, _RPG, L), lambda k: (0, k, 0)),
            pl.BlockSpec((H, bk), lambda k: (0, k)),
            pl.BlockSpec((1, H), lambda k: (0, 0)),
            pl.BlockSpec((C, H), lambda k: (0, 0)),
            pl.BlockSpec((1, C), lambda k: (0, 0)),
        ],
        out_specs=pl.BlockSpec((B, C), lambda k: (0, 0)),
        scratch_shapes=[pltpu.VMEM((B, H), jnp.float32)],
        compiler_params=pltpu.CompilerParams(
            dimension_semantics=("arbitrary",),
            vmem_limit_bytes=44 << 20,
        ),
    )(emb, w1, b1.reshape(1, H), w2, b2.reshape(1, C))


# speedup vs baseline: 1.0109x; 1.0109x over previous
import jax
import jax.numpy as jnp
from jax import lax
from jax.experimental import pallas as pl
from jax.experimental.pallas import tpu as pltpu

_RPG = 56


def _fused_kernel(x_ref, w1_ref, b1_ref, w2_ref, b2_ref, out_ref, acc_ref,
                  *, rows_total):
    k = pl.program_id(0)
    steps = pl.num_programs(0)
    lanes = x_ref.shape[2]
    always_valid = rows_total - (steps - 1) * _RPG

    def sub_dot(t):
        return lax.dot_general(
            x_ref[:, t, :], w1_ref[:, t * lanes:(t + 1) * lanes],
            dimension_numbers=(((1,), (1,)), ((), ())),
            preferred_element_type=jnp.float32)

    acc = sub_dot(0)
    for t in range(1, _RPG):
        p = sub_dot(t)
        if t >= always_valid:
            p = jnp.where(k * _RPG + t < rows_total, p, 0.0)
        acc += p

    @pl.when(k == 0)
    def _init():
        acc_ref[...] = acc

    @pl.when(k > 0)
    def _accum():
        acc_ref[...] += acc

    @pl.when(k == steps - 1)
    def _finalize():
        hidden = acc_ref[...] + b1_ref[...]
        hidden = 1.0 / (1.0 + jnp.exp(-hidden))
        pred = lax.dot_general(
            hidden, w2_ref[...],
            dimension_numbers=(((1,), (1,)), ((), ())),
            preferred_element_type=jnp.float32) + b2_ref[...]
        out_ref[...] = pred.astype(out_ref.dtype)


def kernel(emb, w1, b1, w2, b2):
    import functools
    B, R, L = emb.shape
    H, K = w1.shape
    C = w2.shape[0]
    assert R * L == K, (emb.shape, w1.shape)
    steps = -(-R // _RPG)
    bk = _RPG * L

    return pl.pallas_call(
        functools.partial(_fused_kernel, rows_total=R),
        out_shape=jax.ShapeDtypeStruct((B, C), emb.dtype),
        grid=(steps,),
        in_specs=[
            pl.BlockSpec((B, _RPG, L), lambda k: (0, k, 0)),
            pl.BlockSpec((H, bk), lambda k: (0, k)),
            pl.BlockSpec((1, H), lambda k: (0, 0)),
            pl.BlockSpec((C, H), lambda k: (0, 0)),
            pl.BlockSpec((1, C), lambda k: (0, 0)),
        ],
        out_specs=pl.BlockSpec((B, C), lambda k: (0, 0)),
        scratch_shapes=[pltpu.VMEM((B, H), jnp.float32)],
        compiler_params=pltpu.CompilerParams(
            dimension_semantics=("arbitrary",),
            vmem_limit_bytes=44 << 20,
        ),
    )(emb, w1, b1.reshape(1, H), w2, b2.reshape(1, C))


# speedup vs baseline: 1.0260x; 1.0149x over previous
import functools

import jax
import jax.numpy as jnp
from jax import lax
from jax.experimental import pallas as pl
from jax.experimental.pallas import tpu as pltpu

_RPG = 48


def _fused_kernel(x_ref, w1_ref, b1_ref, w2_ref, b2_ref, out_ref, acc_ref,
                  *, rows_total):
    k = pl.program_id(0)
    steps = pl.num_programs(0)
    lanes = x_ref.shape[2]
    block = steps - 1 - k
    always_valid = rows_total - (steps - 1) * _RPG

    def sub_dot(t):
        return lax.dot_general(
            x_ref[:, t, :], w1_ref[:, t * lanes:(t + 1) * lanes],
            dimension_numbers=(((1,), (1,)), ((), ())),
            preferred_element_type=jnp.float32)

    acc = sub_dot(0)
    for t in range(1, _RPG):
        p = sub_dot(t)
        if t >= always_valid:
            p = jnp.where(block * _RPG + t < rows_total, p, 0.0)
        acc += p

    @pl.when(k == 0)
    def _init():
        acc_ref[...] = acc

    @pl.when(k > 0)
    def _accum():
        acc_ref[...] += acc

    @pl.when(k == steps - 1)
    def _finalize():
        hidden = acc_ref[...] + b1_ref[...]
        hidden = 1.0 / (1.0 + jnp.exp(-hidden))
        pred = lax.dot_general(
            hidden, w2_ref[...],
            dimension_numbers=(((1,), (1,)), ((), ())),
            preferred_element_type=jnp.float32) + b2_ref[...]
        out_ref[...] = pred.astype(out_ref.dtype)


def kernel(emb, w1, b1, w2, b2):
    B, R, L = emb.shape
    H, K = w1.shape
    C = w2.shape[0]
    assert R * L == K, (emb.shape, w1.shape)
    steps = -(-R // _RPG)
    bk = _RPG * L

    return pl.pallas_call(
        functools.partial(_fused_kernel, rows_total=R),
        out_shape=jax.ShapeDtypeStruct((B, C), emb.dtype),
        grid=(steps,),
        in_specs=[
            pl.BlockSpec((B, _RPG, L), lambda k: (0, steps - 1 - k, 0)),
            pl.BlockSpec((H, bk), lambda k: (0, steps - 1 - k)),
            pl.BlockSpec((1, H), lambda k: (0, 0)),
            pl.BlockSpec((C, H), lambda k: (0, 0)),
            pl.BlockSpec((1, C), lambda k: (0, 0)),
        ],
        out_specs=pl.BlockSpec((B, C), lambda k: (0, 0)),
        scratch_shapes=[pltpu.VMEM((B, H), jnp.float32)],
        compiler_params=pltpu.CompilerParams(
            dimension_semantics=("arbitrary",),
            vmem_limit_bytes=44 << 20,
        ),
    )(emb, w1, b1.reshape(1, H), w2, b2.reshape(1, C))


# speedup vs baseline: 1.0482x; 1.0217x over previous
import functools

import jax
import jax.numpy as jnp
from jax import lax
from jax.experimental import pallas as pl
from jax.experimental.pallas import tpu as pltpu

_RPG = 40


def _fused_kernel(x_ref, w1_ref, b1_ref, w2_ref, b2_ref, out_ref, acc_ref,
                  *, rows_total):
    k = pl.program_id(0)
    steps = pl.num_programs(0)
    lanes = x_ref.shape[2]
    block = steps - 1 - k
    always_valid = rows_total - (steps - 1) * _RPG

    def sub_dot(t):
        return lax.dot_general(
            x_ref[:, t, :], w1_ref[:, t * lanes:(t + 1) * lanes],
            dimension_numbers=(((1,), (1,)), ((), ())),
            preferred_element_type=jnp.float32)

    acc = sub_dot(0)
    for t in range(1, _RPG):
        p = sub_dot(t)
        if t >= always_valid:
            p = jnp.where(block * _RPG + t < rows_total, p, 0.0)
        acc += p

    @pl.when(k == 0)
    def _init():
        acc_ref[...] = acc

    @pl.when(k > 0)
    def _accum():
        acc_ref[...] += acc

    @pl.when(k == steps - 1)
    def _finalize():
        hidden = acc_ref[...] + b1_ref[...]
        hidden = 1.0 / (1.0 + jnp.exp(-hidden))
        pred = lax.dot_general(
            hidden, w2_ref[...],
            dimension_numbers=(((1,), (1,)), ((), ())),
            preferred_element_type=jnp.float32) + b2_ref[...]
        out_ref[...] = pred.astype(out_ref.dtype)


def kernel(emb, w1, b1, w2, b2):
    B, R, L = emb.shape
    H, K = w1.shape
    C = w2.shape[0]
    assert R * L == K, (emb.shape, w1.shape)
    steps = -(-R // _RPG)
    bk = _RPG * L

    return pl.pallas_call(
        functools.partial(_fused_kernel, rows_total=R),
        out_shape=jax.ShapeDtypeStruct((B, C), emb.dtype),
        grid=(steps,),
        in_specs=[
            pl.BlockSpec((B, _RPG, L), lambda k: (0, steps - 1 - k, 0)),
            pl.BlockSpec((H, bk), lambda k: (0, steps - 1 - k)),
            pl.BlockSpec((1, H), lambda k: (0, 0)),
            pl.BlockSpec((C, H), lambda k: (0, 0)),
            pl.BlockSpec((1, C), lambda k: (0, 0)),
        ],
        out_specs=pl.BlockSpec((B, C), lambda k: (0, 0)),
        scratch_shapes=[pltpu.VMEM((B, H), jnp.float32)],
        compiler_params=pltpu.CompilerParams(
            dimension_semantics=("arbitrary",),
            vmem_limit_bytes=44 << 20,
        ),
    )(emb, w1, b1.reshape(1, H), w2, b2.reshape(1, C))
